# Initial kernel scaffold; baseline (speedup 1.0000x reference)
import jax
import jax.numpy as jnp
from jax.experimental import pallas as pl


def kernel(x, W):
    raise NotImplementedError("write your pallas kernel here")


# speedup vs baseline: 3.0461x; 3.0461x over previous
import functools

import jax
import jax.numpy as jnp
from jax import lax
from jax.experimental import pallas as pl
from jax.experimental.pallas import tpu as pltpu
from jax.experimental.pallas import tpu_sc as plsc

_NUM_EMB = 197
_DIM = 768
_HALF_COLS = 384
_NUM_SUBCORES = 16


def _broadcast_table_sc(W, batch):
    b_per_tile = batch // _NUM_SUBCORES
    mesh = plsc.VectorSubcoreMesh(core_axis_name="c", subcore_axis_name="s")

    @functools.partial(
        pl.kernel,
        out_type=jax.ShapeDtypeStruct((batch, _NUM_EMB, _DIM), W.dtype),
        mesh=mesh,
        scratch_types=[
            pltpu.VMEM((_NUM_EMB, _HALF_COLS), W.dtype),
            pltpu.SemaphoreType.DMA,
        ],
    )
    def k(w_hbm, out_hbm, w_tile, sem):
        core = lax.axis_index("c")
        sub = lax.axis_index("s")
        c0 = core * _HALF_COLS
        pltpu.sync_copy(w_hbm.at[:, pl.ds(c0, _HALF_COLS)], w_tile)
        base = sub * b_per_tile

        @pl.loop(0, b_per_tile)
        def _(i):
            pltpu.async_copy(
                w_tile, out_hbm.at[base + i, :, pl.ds(c0, _HALF_COLS)], sem
            )

        @pl.loop(0, b_per_tile)
        def _(i):
            pltpu.make_async_copy(
                w_tile, out_hbm.at[base + i, :, pl.ds(c0, _HALF_COLS)], sem
            ).wait()

    return k(W)


def kernel(x, W):
    return _broadcast_table_sc(W, x.shape[0])
